# baseline (speedup 1.0000x reference)
import functools

import jax
import jax.numpy as jnp
from jax import lax
from jax.experimental import pallas as pl
from jax.experimental.pallas import tpu as pltpu
from jax.experimental.pallas import tpu_sc as plsc

_EMBED = 64
_NC = 2
_NS = 16
_NW = _NC * _NS
_BT = 128
_NSR = 2
_NSW = 4


def _sc_gather(t5, P, n_h):
    nbt = t5.shape[1]
    net = _EMBED // 8
    mesh = plsc.VectorSubcoreMesh(core_axis_name="c", subcore_axis_name="s")

    @functools.partial(
        pl.kernel,
        mesh=mesh,
        out_type=jax.ShapeDtypeStruct((n_h, net, nbt, 8, _BT), jnp.float32),
        scratch_types=[
            pltpu.VMEM((n_h // 8, 8, _BT), jnp.int32),
            pltpu.VMEM((_NSR, _BT, _EMBED), jnp.float32),
            pltpu.VMEM((_NSW, net, 8, _BT), jnp.float32),
            pltpu.VMEM_SHARED((8192, _EMBED), jnp.float32),
            [pltpu.SemaphoreType.DMA] * _NSR,
            [pltpu.SemaphoreType.DMA] * _NSW,
        ],
        compiler_params=pltpu.CompilerParams(
            use_tc_tiling_on_sc=False, needs_layout_passes=False
        ),
    )
    def k(t_hbm, P_hbm, out_hbm, idx_v, rows_v, tr_v, table_sh, gsems, wsems):
        sid = lax.axis_index("s")
        wid = sid * _NC + lax.axis_index("c")
        shard = 8192 // _NS
        pltpu.sync_copy(
            P_hbm.at[pl.ds(sid * shard, shard)],
            table_sh.at[pl.ds(sid * shard, shard)],
        )
        pltpu.sync_copy(t_hbm.at[:, wid], idx_v)
        plsc.subcore_barrier()

        lane = lax.iota(jnp.int32, 16)
        bvecs = [lane + jnp.int32(g * 16) for g in range(8)]

        def gather_start(h, r):
            pltpu.async_copy(
                table_sh.at[idx_v.at[h // 8, h % 8]], rows_v.at[r], gsems[r]
            )

        def gather_wait(r):
            pltpu.make_async_copy(
                P_hbm.at[pl.ds(0, _BT)], rows_v.at[r], gsems[r]
            ).wait()

        def write_start(h, u):
            pltpu.async_copy(tr_v.at[u], out_hbm.at[h, :, wid], wsems[u])

        def write_wait(u):
            pltpu.make_async_copy(
                tr_v.at[u], out_hbm.at[0, :, wid], wsems[u]
            ).wait()

        def transpose(r, u):
            rows = rows_v.at[r]

            @pl.loop(0, _EMBED)
            def _t(e):
                evec = jnp.full((16,), e, jnp.int32)
                et = e // 8
                ei = e % 8
                for g in range(8):
                    v = plsc.load_gather(rows, [bvecs[g], evec])
                    tr_v[u, et, ei, pl.ds(g * 16, 16)] = v

        for r in range(_NSR):
            gather_start(r, r)

        @pl.loop(0, n_h, step=_NSR * 2)
        def _body(h0):
            for b in range(_NSR * 2):
                h = h0 + b
                r = b % _NSR
                u = b % _NSW

                gather_wait(r)

                @pl.when(h >= _NSW)
                def _():
                    write_wait(u)

                transpose(r, u)

                @pl.when(h + _NSR < n_h)
                def _():
                    gather_start(h + _NSR, r)

                write_start(h, u)

        for u in range(_NSW):
            write_wait(u)

    return k(t5, P)


def kernel(t, P):
    B, H = t.shape
    t5 = t.T.reshape(H // 8, 8, B // _BT, _BT).transpose(0, 2, 1, 3)
    o5 = _sc_gather(t5, P, H)
    return o5.transpose(2, 4, 0, 1, 3).reshape(B, H, _EMBED)


# speedup vs baseline: 1.7734x; 1.7734x over previous
import functools

import jax
import jax.numpy as jnp
from jax import lax
from jax.experimental import pallas as pl
from jax.experimental.pallas import tpu as pltpu
from jax.experimental.pallas import tpu_sc as plsc

_EMBED = 64
_NC = 2
_NS = 16
_NW = _NC * _NS
_BT = 128
_NSR = 2
_NSW = 4


def _sc_gather(t5, P, n_h):
    nbt = t5.shape[1]
    net = _EMBED // 8
    mesh = plsc.VectorSubcoreMesh(core_axis_name="c", subcore_axis_name="s")

    @functools.partial(
        pl.kernel,
        mesh=mesh,
        out_type=jax.ShapeDtypeStruct((n_h, net, nbt, 8, _BT), jnp.float32),
        scratch_types=[
            pltpu.VMEM((n_h // 8, 8, _BT), jnp.int32),
            pltpu.VMEM((_NSR, _BT, _EMBED), jnp.float32),
            pltpu.VMEM((_NSW, net, 8, _BT), jnp.float32),
            pltpu.VMEM_SHARED((8192, _EMBED), jnp.float32),
            [pltpu.SemaphoreType.DMA] * _NSR,
            [pltpu.SemaphoreType.DMA] * _NSW,
        ],
        compiler_params=pltpu.CompilerParams(
            use_tc_tiling_on_sc=False, needs_layout_passes=False
        ),
    )
    def k(t_hbm, P_hbm, out_hbm, idx_v, rows_v, tr_v, table_sh, gsems, wsems):
        sid = lax.axis_index("s")
        wid = sid * _NC + lax.axis_index("c")
        shard = 8192 // _NS
        pltpu.sync_copy(
            P_hbm.at[pl.ds(sid * shard, shard)],
            table_sh.at[pl.ds(sid * shard, shard)],
        )
        pltpu.sync_copy(t_hbm.at[:, wid], idx_v)
        plsc.subcore_barrier()

        lane = lax.iota(jnp.int32, 16)
        bvecs = [lane + jnp.int32(g * 16) for g in range(8)]

        def gather_start(h, r):
            pltpu.async_copy(
                table_sh.at[idx_v.at[h // 8, h % 8]], rows_v.at[r], gsems[r]
            )

        def gather_wait(r):
            pltpu.make_async_copy(
                P_hbm.at[pl.ds(0, _BT)], rows_v.at[r], gsems[r]
            ).wait()

        def write_start(h, u):
            pltpu.async_copy(tr_v.at[u], out_hbm.at[h, :, wid], wsems[u])

        def write_wait(u):
            pltpu.make_async_copy(
                tr_v.at[u], out_hbm.at[0, :, wid], wsems[u]
            ).wait()

        def transpose(r, u):
            rows = rows_v.at[r]

            @plsc.parallel_loop(0, _EMBED, unroll=8)
            def _t(e):
                evec = jnp.full((16,), e, jnp.int32)
                et = e // 8
                ei = e % 8
                for g in range(8):
                    v = plsc.load_gather(rows, [bvecs[g], evec])
                    tr_v[u, et, ei, pl.ds(g * 16, 16)] = v

        for r in range(_NSR):
            gather_start(r, r)

        @pl.loop(0, n_h, step=_NSR * 2)
        def _body(h0):
            for b in range(_NSR * 2):
                h = h0 + b
                r = b % _NSR
                u = b % _NSW

                gather_wait(r)

                @pl.when(h >= _NSW)
                def _():
                    write_wait(u)

                transpose(r, u)

                @pl.when(h + _NSR < n_h)
                def _():
                    gather_start(h + _NSR, r)

                write_start(h, u)

        for u in range(_NSW):
            write_wait(u)

    return k(t5, P)


def kernel(t, P):
    B, H = t.shape
    t5 = t.T.reshape(H // 8, 8, B // _BT, _BT).transpose(0, 2, 1, 3)
    o5 = _sc_gather(t5, P, H)
    return o5.transpose(2, 4, 0, 1, 3).reshape(B, H, _EMBED)
